# baseline (speedup 1.0000x reference)
import functools

import jax
import jax.numpy as jnp
from jax import lax
from jax.experimental import pallas as pl
from jax.experimental.pallas import tpu as pltpu
from jax.experimental.pallas import tpu_sc as plsc

NC = 2
NS = 16
L = 16
NW = NC * NS
B = 16384
D = 64
BPW = B // NW
CHUNK = 128
NCHUNK = BPW // CHUNK
NBLK = BPW // L
NUM_CAT = 1000
NUM_AUTH = 100000
NUM_TIT = 1000000
DRAIN = 16


def _body(cidx, aidx, tidx, sent, catf, authf, titf, wb, out,
          cidx_v, aidx_v, tidx_v, sent_v, wb_v, idx2, gbuf, acc_v,
          sem_s, sem_g):
  wid = lax.axis_index("s") * NC + lax.axis_index("c")
  base = wid * BPW

  stage = []
  for j in range(NCHUNK):
    off = base + CHUNK * j
    stage.append(pltpu.async_copy(cidx.at[pl.ds(off, CHUNK)], cidx_v.at[j], sem_s))
    stage.append(pltpu.async_copy(aidx.at[pl.ds(off, CHUNK)], aidx_v.at[j], sem_s))
    stage.append(pltpu.async_copy(tidx.at[pl.ds(off, CHUNK)], tidx_v.at[j], sem_s))
  stage.append(pltpu.async_copy(sent.at[pl.ds(base, BPW)], sent_v, sem_s))
  stage.append(pltpu.async_copy(wb, wb_v, sem_s))
  for h in stage:
    h.wait()

  wsv = (wb_v[pl.ds(192, L)] + wb_v[pl.ds(208, L)]) + (
      wb_v[pl.ds(224, L)] + wb_v[pl.ds(240, L)])
  ws_sum = jnp.sum(wsv)
  b_s = jnp.sum(wb_v[pl.ds(256, L)])

  def init_blk(i, carry):
    r0 = i * L
    acc_v[pl.ds(r0, L)] = sent_v[pl.ds(r0, L)] * ws_sum + b_s
    return carry
  lax.fori_loop(0, NBLK, init_blk, 0)

  for t, (tab, idx_v, v) in enumerate((
      (catf, cidx_v, NUM_CAT), (authf, aidx_v, NUM_AUTH),
      (titf, tidx_v, NUM_TIT))):

    def build(c, carry):
      cv = c * v
      for j in range(NCHUNK):
        for s in range(CHUNK // L):
          idx2[c, j, pl.ds(L * s, L)] = idx_v[j, pl.ds(L * s, L)] + cv
      return carry
    lax.fori_loop(0, D, build, 0)

    pend = []
    for c in range(D):
      for j in range(NCHUNK):
        pend.append(pltpu.async_copy(
            tab.at[idx2.at[c].at[j]],
            gbuf.at[c].at[pl.ds(CHUNK * j, CHUNK)], sem_g))
        if len(pend) == DRAIN:
          for h in pend:
            h.wait()
          pend = []
    for h in pend:
      h.wait()

    wt = [wb_v[pl.ds(D * t + L * k, L)] for k in range(4)]

    def blk(i, carry):
      r0 = i * L
      p0 = acc_v[pl.ds(r0, L)]
      p1 = jnp.zeros((L,), jnp.float32)
      p2 = jnp.zeros((L,), jnp.float32)
      p3 = jnp.zeros((L,), jnp.float32)
      ps = [p0, p1, p2, p3]
      for c in range(D):
        ps[c % 4] = ps[c % 4] + gbuf[c, pl.ds(r0, L)] * wt[c // L][c % L]
      acc_v[pl.ds(r0, L)] = (ps[0] + ps[1]) + (ps[2] + ps[3])
      return carry
    lax.fori_loop(0, NBLK, blk, 0)

  pltpu.sync_copy(acc_v, out.at[pl.ds(base, BPW)])


@functools.cache
def _build():
  mesh = plsc.VectorSubcoreMesh(
      core_axis_name="c", subcore_axis_name="s", num_cores=NC, num_subcores=NS)
  return pl.kernel(
      _body,
      out_type=jax.ShapeDtypeStruct((B,), jnp.float32),
      mesh=mesh,
      compiler_params=pltpu.CompilerParams(
          needs_layout_passes=False, use_tc_tiling_on_sc=False),
      scratch_types=[
          pltpu.VMEM((NCHUNK, CHUNK), jnp.int32),
          pltpu.VMEM((NCHUNK, CHUNK), jnp.int32),
          pltpu.VMEM((NCHUNK, CHUNK), jnp.int32),
          pltpu.VMEM((BPW,), jnp.float32),
          pltpu.VMEM((272,), jnp.float32),
          pltpu.VMEM((D, NCHUNK, CHUNK), jnp.int32),
          pltpu.VMEM((D, BPW), jnp.float32),
          pltpu.VMEM((BPW,), jnp.float32),
          pltpu.SemaphoreType.DMA,
          pltpu.SemaphoreType.DMA,
      ],
  )


def kernel(category_indices, author_indices, title_indices, sentiment_scores,
           category_table, author_table, title_table, linear_w, linear_b):
  wb = jnp.concatenate([
      linear_w.reshape(-1), linear_b.reshape(-1),
      jnp.zeros((15,), jnp.float32)])
  out = _build()(category_indices, author_indices, title_indices,
                 sentiment_scores, category_table.T.reshape(-1),
                 author_table.T.reshape(-1), title_table.T.reshape(-1), wb)
  return out.reshape(B, 1)


# speedup vs baseline: 8.6713x; 8.6713x over previous
import functools

import jax
import jax.numpy as jnp
from jax import lax
from jax.experimental import pallas as pl
from jax.experimental.pallas import tpu as pltpu
from jax.experimental.pallas import tpu_sc as plsc

NC = 2
NS = 16
L = 16
NW = NC * NS
B = 16384
D = 64
DP = 2 * D
BPW = B // NW
CHUNK = 128
NCHUNK = BPW // CHUNK
NBLK = BPW // L


def _body(cidx, aidx, tidx, sent, cat, auth, title, wb, out,
          cidx_v, aidx_v, tidx_v, sent_v, wb_v, rows, acc_v, sem_s, sem_g):
  wid = lax.axis_index("s") * NC + lax.axis_index("c")
  base = wid * BPW

  stage = []
  for j in range(NCHUNK):
    off = base + CHUNK * j
    stage.append(pltpu.async_copy(cidx.at[pl.ds(off, CHUNK)], cidx_v.at[j], sem_s))
    stage.append(pltpu.async_copy(aidx.at[pl.ds(off, CHUNK)], aidx_v.at[j], sem_s))
    stage.append(pltpu.async_copy(tidx.at[pl.ds(off, CHUNK)], tidx_v.at[j], sem_s))
  stage.append(pltpu.async_copy(sent.at[pl.ds(base, BPW)], sent_v, sem_s))
  stage.append(pltpu.async_copy(wb, wb_v, sem_s))
  for h in stage:
    h.wait()

  wsv = (wb_v[pl.ds(192, L)] + wb_v[pl.ds(208, L)]) + (
      wb_v[pl.ds(224, L)] + wb_v[pl.ds(240, L)])
  ws_sum = jnp.sum(wsv)
  b_s = jnp.sum(wb_v[pl.ds(256, L)])

  def init_blk(i, carry):
    r0 = i * L
    acc_v[pl.ds(r0, L)] = sent_v[pl.ds(r0, L)] * ws_sum + b_s
    return carry
  lax.fori_loop(0, NBLK, init_blk, 0)

  lane = lax.iota(jnp.int32, L)

  for t, (table, idx_v) in enumerate(
      ((cat, cidx_v), (auth, aidx_v), (title, tidx_v))):
    gath = [
        pltpu.async_copy(table.at[idx_v.at[j]],
                         rows.at[pl.ds(CHUNK * j, CHUNK)], sem_g)
        for j in range(NCHUNK)
    ]
    for h in gath:
      h.wait()

    wks = [wb_v[pl.ds(D * t + L * k, L)] for k in range(4)]

    def blk(i, carry):
      r0 = i * L
      dots = jnp.zeros((L,), jnp.float32)
      for j in range(L):
        row = r0 + j
        pa = rows[row, pl.ds(0, L)] * wks[0]
        pb = rows[row, pl.ds(L, L)] * wks[1]
        pa = pa + rows[row, pl.ds(2 * L, L)] * wks[2]
        pb = pb + rows[row, pl.ds(3 * L, L)] * wks[3]
        sj = jnp.sum(pa + pb)
        dots = jnp.where(lane == j, sj, dots)
      acc_v[pl.ds(r0, L)] = acc_v[pl.ds(r0, L)] + dots
      return carry

    lax.fori_loop(0, NBLK, blk, 0)

  pltpu.sync_copy(acc_v, out.at[pl.ds(base, BPW)])


@functools.cache
def _build():
  mesh = plsc.VectorSubcoreMesh(
      core_axis_name="c", subcore_axis_name="s", num_cores=NC, num_subcores=NS)
  return pl.kernel(
      _body,
      out_type=jax.ShapeDtypeStruct((B,), jnp.float32),
      mesh=mesh,
      compiler_params=pltpu.CompilerParams(
          needs_layout_passes=False, use_tc_tiling_on_sc=False),
      scratch_types=[
          pltpu.VMEM((NCHUNK, CHUNK), jnp.int32),
          pltpu.VMEM((NCHUNK, CHUNK), jnp.int32),
          pltpu.VMEM((NCHUNK, CHUNK), jnp.int32),
          pltpu.VMEM((BPW,), jnp.float32),
          pltpu.VMEM((272,), jnp.float32),
          pltpu.VMEM((BPW, DP), jnp.float32),
          pltpu.VMEM((BPW,), jnp.float32),
          pltpu.SemaphoreType.DMA,
          pltpu.SemaphoreType.DMA,
      ],
  )


def _pad128(x):
  return jnp.pad(x, ((0, 0), (0, D)))


def kernel(category_indices, author_indices, title_indices, sentiment_scores,
           category_table, author_table, title_table, linear_w, linear_b):
  wb = jnp.concatenate([
      linear_w.reshape(-1), linear_b.reshape(-1),
      jnp.zeros((15,), jnp.float32)])
  out = _build()(category_indices, author_indices, title_indices,
                 sentiment_scores, _pad128(category_table),
                 _pad128(author_table), _pad128(title_table), wb)
  return out.reshape(B, 1)
